# baseline (speedup 1.0000x reference)
import functools

import jax
import jax.numpy as jnp
from jax import lax
from jax.experimental import pallas as pl
from jax.experimental.pallas import tpu as pltpu
from jax.experimental.pallas import tpu_sc as plsc

NE = 10
LANES = 512
SUB = 8


def _gl_kernel(n, posF_ref, posT_ref, x2F_ref, x2T_ref, uT_ref, inv_ref,
               idx_ref, w_ref, ps_ref):
    ntile = n // 128
    j0 = pl.program_id(1)

    pall = posF_ref[0]
    prow = posT_ref[0]
    dotT = lax.dot_general(pall, prow, (((1,), (0,)), ((), ())),
                           precision='default',
                           preferred_element_type=jnp.float32)
    sqT = (x2F_ref[0] + x2T_ref[0]) - 2.0 * dotT
    cdT = jnp.sqrt(jnp.maximum(sqT, 1e-12))
    w2 = jnp.exp(-(cdT * inv_ref[0, 0]))
    jcol = lax.broadcasted_iota(jnp.int32, (n, LANES), 0)
    rglob = j0 * LANES + lax.broadcasted_iota(jnp.int32, (n, LANES), 1)
    wm = jnp.where(jcol == rglob, 0.0, w2)
    w_ref[0] = wm

    m = jnp.max(wm, axis=0)[None, None, :]

    ps_ref[...] = jnp.exp(wm.reshape(ntile, 128, LANES) - m)

    def _sm(v, acc):
        return acc + ps_ref[v // 16, pl.ds(SUB * (v % 16), SUB), :]
    sacc = lax.fori_loop(1, ntile * 16, _sm, ps_ref[0, 0:SUB, :])
    t = sacc[0:4, :] + sacc[4:8, :]
    t = t[0:2, :] + t[2:4, :]
    s = (t[0:1, :] + t[1:2, :])[None]

    ps_ref[...] = ps_ref[...] / s

    def _scan(i, _):
        ps_ref[:, pl.ds(i, 1), :] = (ps_ref[:, pl.ds(i - 1, 1), :]
                                     + ps_ref[:, pl.ds(i, 1), :])
        return 0
    lax.fori_loop(1, 128, _scan, 0)

    lasts = ps_ref[:, 127:128, :]
    offs = [jnp.zeros((1, 1, LANES), jnp.float32)]
    acc = lasts[0:1]
    for tt in range(1, ntile):
        offs.append(acc)
        if tt < ntile - 1:
            acc = acc + lasts[tt:tt + 1]
    ps_ref[...] = ps_ref[...] + jnp.concatenate(offs, axis=0)

    psv = ps_ref[...]
    for k in range(NE):
        ue = uT_ref[0, k:k + 1, :][None]
        cnt = jnp.sum(jnp.where(psv < ue, 1.0, 0.0), axis=(0, 1))
        idxv = jnp.clip(cnt.astype(jnp.int32), 0, n - 1)[None, :]
        idx_ref[0, k:k + 1, :] = idxv


def _make_sc_gather(total, n_elems):
    info = plsc.get_sparse_core_info()
    nw = info.num_cores * info.num_subcores
    perw = total // nw
    mesh = plsc.VectorSubcoreMesh(core_axis_name="c", subcore_axis_name="s")

    @functools.partial(
        pl.kernel, mesh=mesh,
        out_type=jax.ShapeDtypeStruct((total,), jnp.float32),
        scratch_types=[
            pltpu.VMEM((perw,), jnp.int32),
            pltpu.VMEM((perw,), jnp.float32),
            pltpu.SemaphoreType.DMA,
        ],
    )
    def _gather(wflat_hbm, fidx_hbm, out_hbm, idx_v, vals_v, sem):
        wid = lax.axis_index("s") * info.num_cores + lax.axis_index("c")
        base = wid * perw
        pltpu.sync_copy(fidx_hbm.at[pl.ds(base, perw)], idx_v)
        pltpu.async_copy(wflat_hbm.at[idx_v], vals_v, sem).wait()
        pltpu.sync_copy(vals_v, out_hbm.at[pl.ds(base, perw)])

    return _gather


def kernel(pos, batch, eps):
    B = 4
    N, d = pos.shape
    n = N // B
    pos_b = pos.reshape(B, n, d)
    posT = jnp.transpose(pos_b, (0, 2, 1))
    p0, p1, p2 = pos_b[..., 0], pos_b[..., 1], pos_b[..., 2]
    x2 = (p0 * p0 + p2 * p2) + p1 * p1
    x2F = x2[:, :, None]
    x2T = x2[:, None, :]
    inv = (1.0 / (eps[0] ** 2)).reshape(1, 1).astype(jnp.float32)
    u = jax.random.uniform(jax.random.key(42), (B, n, NE), dtype=jnp.float32)
    uT = jnp.transpose(u, (0, 2, 1))

    nb = n // LANES
    grid = (B, nb)
    idxT, wT = pl.pallas_call(
        functools.partial(_gl_kernel, n),
        grid=grid,
        in_specs=[
            pl.BlockSpec((1, n, 3), lambda b, j: (b, 0, 0)),
            pl.BlockSpec((1, 3, LANES), lambda b, j: (b, 0, j)),
            pl.BlockSpec((1, n, 1), lambda b, j: (b, 0, 0)),
            pl.BlockSpec((1, 1, LANES), lambda b, j: (b, 0, j)),
            pl.BlockSpec((1, NE, LANES), lambda b, j: (b, 0, j)),
            pl.BlockSpec((1, 1), lambda b, j: (0, 0)),
        ],
        out_specs=[
            pl.BlockSpec((1, NE, LANES), lambda b, j: (b, 0, j)),
            pl.BlockSpec((1, n, LANES), lambda b, j: (b, 0, j)),
        ],
        out_shape=[
            jax.ShapeDtypeStruct((B, NE, n), jnp.int32),
            jax.ShapeDtypeStruct((B, n, n), jnp.float32),
        ],
        scratch_shapes=[
            pltpu.VMEM((n // 128, 128, LANES), jnp.float32),
        ],
    )(pos_b, posT, x2F, x2T, uT, inv)

    riota = jnp.arange(n, dtype=jnp.int32)[None, None, :]
    fidx = (jnp.int32(n * n) * jnp.arange(B, dtype=jnp.int32)[:, None, None]
            + idxT * jnp.int32(n) + riota).reshape(-1)
    gather = _make_sc_gather(B * NE * n, n * n * B)
    attr_flat = gather(wT.reshape(-1), fidx)

    idx_target = jnp.transpose(idxT, (0, 2, 1))
    edge_attr = jnp.transpose(attr_flat.reshape(B, NE, n),
                              (0, 2, 1)).reshape(-1)
    offsets = (batch.reshape(B, n)[:, 0] * n)[:, None, None]
    idx_src = jnp.broadcast_to(jnp.arange(n)[:, None], (n, NE))
    ei_src = (idx_src[None, :, :] + offsets).reshape(-1)
    ei_dst = (idx_target + offsets).reshape(-1)
    edge_index = jnp.stack([ei_src, ei_dst], axis=0)
    return edge_index, edge_attr


# speedup vs baseline: 1.2607x; 1.2607x over previous
import functools

import jax
import jax.numpy as jnp
from jax import lax
from jax.experimental import pallas as pl
from jax.experimental.pallas import tpu as pltpu

NE = 10
LANES = 512
SUB = 8


def _gl_kernel(n, posF_ref, posT_ref, x2F_ref, x2T_ref, uT_ref, inv_ref,
               idx_ref, attr_ref, w_ref, ps_ref):
    ntile = n // 128
    j0 = pl.program_id(1)

    pall = posF_ref[0]
    prow = posT_ref[0]
    dotT = lax.dot_general(pall, prow, (((1,), (0,)), ((), ())),
                           precision='default',
                           preferred_element_type=jnp.float32)
    sqT = (x2F_ref[0] + x2T_ref[0]) - 2.0 * dotT
    cdT = jnp.sqrt(jnp.maximum(sqT, 1e-12))
    w2 = jnp.exp(-(cdT * inv_ref[0, 0]))
    jcol = lax.broadcasted_iota(jnp.int32, (n, LANES), 0)
    rglob = j0 * LANES + lax.broadcasted_iota(jnp.int32, (n, LANES), 1)
    wm = jnp.where(jcol == rglob, 0.0, w2)
    w_ref[...] = wm.reshape(ntile, 128, LANES)

    m = jnp.max(wm, axis=0)[None, None, :]

    ps_ref[...] = jnp.exp(w_ref[...] - m)

    sacc = ps_ref[0, 0:SUB, :]
    for v in range(1, ntile * 16):
        sacc = sacc + ps_ref[v // 16, SUB * (v % 16):SUB * (v % 16) + SUB, :]
    t = sacc[0:4, :] + sacc[4:8, :]
    t = t[0:2, :] + t[2:4, :]
    s = (t[0:1, :] + t[1:2, :])[None]

    ps_ref[...] = ps_ref[...] / s

    for tt in range(ntile):
        prev = ps_ref[tt, 0:1, :]
        for r in range(1, 128):
            prev = prev + ps_ref[tt, r:r + 1, :]
            ps_ref[tt, r:r + 1, :] = prev

    lasts = ps_ref[:, 127:128, :]
    offs = [jnp.zeros((1, 1, LANES), jnp.float32)]
    acc = lasts[0:1]
    for tt in range(1, ntile):
        offs.append(acc)
        if tt < ntile - 1:
            acc = acc + lasts[tt:tt + 1]
    ps_ref[...] = ps_ref[...] + jnp.concatenate(offs, axis=0)

    coliota = (128 * lax.broadcasted_iota(jnp.int32, (ntile, 128, LANES), 0)
               + lax.broadcasted_iota(jnp.int32, (ntile, 128, LANES), 1))
    psv = ps_ref[...]
    wv = w_ref[...]
    for k in range(NE):
        ue = uT_ref[0, k:k + 1, :][None]
        cnt = jnp.sum(jnp.where(psv < ue, 1.0, 0.0), axis=(0, 1))
        idxv = jnp.clip(cnt.astype(jnp.int32), 0, n - 1)[None, :]
        attrv = jnp.sum(jnp.where(coliota == idxv[None], wv, 0.0), axis=(0, 1))
        idx_ref[0, k:k + 1, :] = idxv
        attr_ref[0, k:k + 1, :] = attrv[None, :]


def kernel(pos, batch, eps):
    B = 4
    N, d = pos.shape
    n = N // B
    pos_b = pos.reshape(B, n, d)
    posT = jnp.transpose(pos_b, (0, 2, 1))
    p0, p1, p2 = pos_b[..., 0], pos_b[..., 1], pos_b[..., 2]
    x2 = (p0 * p0 + p2 * p2) + p1 * p1
    x2F = x2[:, :, None]
    x2T = x2[:, None, :]
    inv = (1.0 / (eps[0] ** 2)).reshape(1, 1).astype(jnp.float32)
    u = jax.random.uniform(jax.random.key(42), (B, n, NE), dtype=jnp.float32)
    uT = jnp.transpose(u, (0, 2, 1))

    nb = n // LANES
    grid = (B, nb)
    idxT, attrT = pl.pallas_call(
        functools.partial(_gl_kernel, n),
        grid=grid,
        in_specs=[
            pl.BlockSpec((1, n, 3), lambda b, j: (b, 0, 0)),
            pl.BlockSpec((1, 3, LANES), lambda b, j: (b, 0, j)),
            pl.BlockSpec((1, n, 1), lambda b, j: (b, 0, 0)),
            pl.BlockSpec((1, 1, LANES), lambda b, j: (b, 0, j)),
            pl.BlockSpec((1, NE, LANES), lambda b, j: (b, 0, j)),
            pl.BlockSpec((1, 1), lambda b, j: (0, 0)),
        ],
        out_specs=[
            pl.BlockSpec((1, NE, LANES), lambda b, j: (b, 0, j)),
            pl.BlockSpec((1, NE, LANES), lambda b, j: (b, 0, j)),
        ],
        out_shape=[
            jax.ShapeDtypeStruct((B, NE, n), jnp.int32),
            jax.ShapeDtypeStruct((B, NE, n), jnp.float32),
        ],
        scratch_shapes=[
            pltpu.VMEM((n // 128, 128, LANES), jnp.float32),
            pltpu.VMEM((n // 128, 128, LANES), jnp.float32),
        ],
    )(pos_b, posT, x2F, x2T, uT, inv)

    idx_target = jnp.transpose(idxT, (0, 2, 1))
    edge_attr = jnp.transpose(attrT, (0, 2, 1)).reshape(-1)
    offsets = (batch.reshape(B, n)[:, 0] * n)[:, None, None]
    idx_src = jnp.broadcast_to(jnp.arange(n)[:, None], (n, NE))
    ei_src = (idx_src[None, :, :] + offsets).reshape(-1)
    ei_dst = (idx_target + offsets).reshape(-1)
    edge_index = jnp.stack([ei_src, ei_dst], axis=0)
    return edge_index, edge_attr
